# baseline (speedup 1.0000x reference)
import functools

import jax
import jax.numpy as jnp
import numpy as np
from jax import lax
from jax.experimental import pallas as pl
from jax.experimental.pallas import tpu as pltpu
from jax.experimental.pallas import tpu_sc as plsc

NC = 2
NS = 16
L = 16
NW = NC * NS
K = 8
HP = 64
VB = 512


@functools.lru_cache(maxsize=None)
def _build_fmt(B, H, V, E):
  NBLK = V // VB
  FULL = NBLK // NW
  TAIL = V - NBLK * VB
  assert B // NW == VB and E == 2 * 2 * L and H <= HP
  mesh = plsc.VectorSubcoreMesh(core_axis_name="c", subcore_axis_name="s")

  @functools.partial(
      pl.kernel,
      mesh=mesh,
      compiler_params=pltpu.CompilerParams(
          use_tc_tiling_on_sc=False, needs_layout_passes=False),
      out_type=(jax.ShapeDtypeStruct((V, E), jnp.bfloat16),
                jax.ShapeDtypeStruct((B, HP), jnp.int32)),
      scratch_types=[
          pltpu.VMEM((E, VB), jnp.int32),
          pltpu.VMEM((E, VB), jnp.int32),
          pltpu.VMEM((VB, E), jnp.bfloat16),
          pltpu.VMEM((VB, E), jnp.bfloat16),
          pltpu.VMEM((VB, HP), jnp.int32),
          pltpu.SemaphoreType.DMA,
          pltpu.SemaphoreType.DMA,
          pltpu.SemaphoreType.DMA,
          pltpu.SemaphoreType.DMA,
      ],
  )
  def body(kT_hbm, tT_hbm, tbf_hbm, krm_hbm, in0, in1, tout0, tout1, kout,
           sin0, sin1, sout0, sout1):
    wid = lax.axis_index("s") * NC + lax.axis_index("c")
    ins = (in0, in1)
    touts = (tout0, tout1)
    sins = (sin0, sin1)
    souts = (sout0, sout1)
    lane = jnp.arange(L, dtype=jnp.int32)

    b0 = wid * VB

    def kzero(g, carry):
      for r in range(H, HP):
        in0[r, pl.ds(g * L, L)] = jnp.zeros((L,), jnp.int32)
      return carry

    lax.fori_loop(0, VB // L, kzero, 0)
    pltpu.sync_copy(kT_hbm.at[:, pl.ds(b0, VB)], in0.at[pl.ds(0, H)])

    def ktrans(vv, carry):
      for grp in range(HP // L):
        g = plsc.load_gather(in0, [lane + grp * L, jnp.full((L,), vv)])
        kout[vv, pl.ds(grp * L, L)] = g
      return carry

    lax.fori_loop(0, VB, ktrans, 0)
    pltpu.sync_copy(kout, krm_hbm.at[pl.ds(b0, VB)])

    def issue_in(blk, half):
      pltpu.async_copy(
          tT_hbm.at[:, pl.ds((blk * NW + wid) * VB, VB)], ins[half],
          sins[half])

    def wait_in(half):
      pltpu.make_async_copy(
          tT_hbm.at[:, pl.ds(wid * VB, VB)], ins[half], sins[half]).wait()

    def wait_out(half):
      pltpu.make_async_copy(
          touts[half], tbf_hbm.at[pl.ds(wid * VB, VB)], souts[half]).wait()

    def transpose_block(half, nrows):
      def tt_row(vv, carry):
        for grp in range(2):
          sel = jnp.full((L,), vv)
          ev = plsc.bitcast(
              plsc.load_gather(ins[half], [lane * 2 + grp * 2 * L, sel]),
              jnp.float32)
          od = plsc.bitcast(
              plsc.load_gather(ins[half], [lane * 2 + 1 + grp * 2 * L, sel]),
              jnp.float32)
          touts[half][vv, pl.ds(grp * 2 * L, 2 * L)] = plsc.pack(
              ev, od, format=plsc.PackFormat.INTERLEAVED)
        return carry

      lax.fori_loop(0, nrows, tt_row, 0)

    issue_in(0, 0)
    issue_in(1, 1)

    def pair(t, carry):
      for half in range(2):
        blk = 2 * t + half
        v0 = (blk * NW + wid) * VB
        wait_in(half)

        @pl.when(blk >= 2)
        def _(half=half):
          wait_out(half)

        transpose_block(half, VB)
        nxt = blk + 2

        @pl.when(nxt <= FULL - 1)
        def _(nxt=nxt, half=half):
          issue_in(nxt, half)

        pltpu.async_copy(
            touts[half], tbf_hbm.at[pl.ds(v0, VB)], souts[half])
      return carry

    lax.fori_loop(0, (FULL - 1) // 2, pair, 0)
    blk = FULL - 1
    wait_in(blk % 2)
    wait_out(blk % 2)
    transpose_block(blk % 2, VB)
    pltpu.async_copy(
        touts[blk % 2],
        tbf_hbm.at[pl.ds((blk * NW + wid) * VB, VB)], souts[blk % 2])
    wait_out(blk % 2)
    wait_out(1 - blk % 2)

    @pl.when(wid < NBLK - FULL * NW)
    def _():
      v0 = (FULL * NW + wid) * VB
      pltpu.sync_copy(tT_hbm.at[:, pl.ds(v0, VB)], ins[0])
      transpose_block(0, VB)
      pltpu.sync_copy(touts[0], tbf_hbm.at[pl.ds(v0, VB)])

    if TAIL:
      @pl.when(wid == NS)
      def _():
        v0 = NBLK * VB
        pltpu.sync_copy(tT_hbm.at[:, pl.ds(v0, TAIL)],
                        ins[1].at[:, pl.ds(0, TAIL)])
        transpose_block(1, TAIL)
        pltpu.sync_copy(touts[1].at[pl.ds(0, TAIL)],
                        tbf_hbm.at[pl.ds(v0, TAIL)])

  return body


@functools.lru_cache(maxsize=None)
def _build_gather(B, V, E):
  RPT = B // NW
  NG = RPT
  EW = E // 2
  assert NG % K == 0 and EW % L == 0
  mesh = plsc.VectorSubcoreMesh(core_axis_name="c", subcore_axis_name="s")

  @functools.partial(
      pl.kernel,
      mesh=mesh,
      compiler_params=pltpu.CompilerParams(
          use_tc_tiling_on_sc=False, needs_layout_passes=False),
      out_type=jax.ShapeDtypeStruct((B, E), jnp.float32),
      scratch_types=[
          pltpu.VMEM((RPT, HP), jnp.int32),
          pltpu.VMEM((RPT, E), jnp.float32),
          pltpu.VMEM((RPT + L,), jnp.float32),
      ] + [pltpu.VMEM((HP, E), jnp.bfloat16) for _ in range(K)]
        + [pltpu.SemaphoreType.DMA for _ in range(K)],
  )
  def body(k_hbm, len_hbm, table_hbm, out_hbm, idx_v, out_v, len_v, *ring):
    rows = ring[:K]
    sems = ring[K:]
    wid = lax.axis_index("s") * NC + lax.axis_index("c")
    pltpu.sync_copy(k_hbm.at[pl.ds(wid * RPT, RPT)], idx_v)
    pltpu.sync_copy(len_hbm.at[pl.ds(wid * RPT, RPT)],
                    len_v.at[pl.ds(0, RPT)])

    for j in range(K):
      pltpu.async_copy(table_hbm.at[idx_v.at[j]], rows[j], sems[j])

    hi_mask = jnp.full((L,), jnp.int32(-65536))

    def outer(it, carry):
      g0 = it * K
      for b in range(K):
        i = g0 + b
        pltpu.make_async_copy(
            table_hbm.at[idx_v.at[i]], rows[b], sems[b]).wait()
        ln = len_v[pl.ds(i, L)][0]

        def acc_step(l, accs, _rows=rows[b]):
          new = []
          for w in range(EW // L):
            wv = plsc.bitcast(_rows[l, pl.ds(w * 2 * L, 2 * L)], jnp.int32)
            lo = plsc.bitcast(lax.shift_left(wv, 16), jnp.float32)
            hi = plsc.bitcast(lax.bitwise_and(wv, hi_mask), jnp.float32)
            new.append(accs[2 * w] + lo)
            new.append(accs[2 * w + 1] + hi)
          return tuple(new)

        accs = lax.fori_loop(
            0, HP, acc_step,
            tuple(jnp.zeros((L,), jnp.float32) for _ in range(2 * (EW // L))))
        for g in range(2 * (EW // L)):
          out_v[i, pl.ds(g * L, L)] = accs[g] / ln
        nxt = i + K

        @pl.when(nxt < NG)
        def _(b=b, nxt=nxt):
          pltpu.async_copy(table_hbm.at[idx_v.at[nxt]], rows[b], sems[b])
      return carry

    lax.fori_loop(0, NG // K, outer, 0)
    pltpu.sync_copy(out_v, out_hbm.at[pl.ds(wid * RPT, RPT)])

  return body


def kernel(k, lengths, table):
  B, H = k.shape
  V, E = table.shape
  kT = jnp.swapaxes(k, 0, 1)
  tT = jnp.swapaxes(jax.lax.bitcast_convert_type(table, jnp.int32), 0, 1)
  table_bf, k_rm = _build_fmt(B, H, V, E)(kT, tT)
  out_s = _build_gather(B, V, E)(k_rm, lengths, table_bf)
  perm = np.empty((E,), np.int32)
  for c in range(E):
    w_blk, r = divmod(c, 2 * L)
    perm[c] = w_blk * 2 * L + (r % 2) * L + r // 2
  return out_s[:, perm]


# speedup vs baseline: 12.4075x; 12.4075x over previous
import functools

import jax
import jax.numpy as jnp
from jax import lax
from jax.experimental import pallas as pl
from jax.experimental.pallas import tpu as pltpu
from jax.experimental.pallas import tpu_sc as plsc

NC = 2
NS = 16
L = 16
NW = NC * NS
K = 8
U = 5


@functools.lru_cache(maxsize=None)
def _build(B, H, V, E):
  RPT = B // NW
  NG = RPT
  assert NG % K == 0 and H % U == 0 and H <= 128
  mesh = plsc.VectorSubcoreMesh(core_axis_name="c", subcore_axis_name="s")

  @functools.partial(
      pl.kernel,
      mesh=mesh,
      compiler_params=pltpu.CompilerParams(use_tc_tiling_on_sc=False),
      out_type=jax.ShapeDtypeStruct((B, E), jnp.float32),
      scratch_types=[
          pltpu.VMEM((RPT, H), jnp.int32),
          pltpu.VMEM((RPT, E), jnp.float32),
          pltpu.VMEM((RPT + L,), jnp.float32),
      ] + [pltpu.VMEM((H, E), jnp.float32) for _ in range(K)]
        + [pltpu.SemaphoreType.DMA for _ in range(K)],
  )
  def body(k_hbm, len_hbm, table_hbm, out_hbm, idx_v, out_v, len_v, *ring):
    rows = ring[:K]
    sems = ring[K:]
    wid = lax.axis_index("s") * NC + lax.axis_index("c")
    pltpu.sync_copy(k_hbm.at[pl.ds(wid * RPT, RPT)], idx_v)
    pltpu.sync_copy(len_hbm.at[pl.ds(wid * RPT, RPT)],
                    len_v.at[pl.ds(0, RPT)])

    for j in range(K):
      pltpu.async_copy(table_hbm.at[idx_v.at[j]], rows[j], sems[j])

    def outer(it, carry):
      g0 = it * K
      for b in range(K):
        i = g0 + b
        pltpu.make_async_copy(
            table_hbm.at[idx_v.at[i]], rows[b], sems[b]).wait()
        ln = len_v[pl.ds(i, L)][0]

        def acc_step(t, accs, _rows=rows[b]):
          for u in range(U):
            j = t * U + u
            accs = tuple(
                accs[g] + _rows[j, pl.ds(g * L, L)] for g in range(E // L))
          return accs

        accs = lax.fori_loop(
            0, H // U, acc_step,
            tuple(jnp.zeros((L,), jnp.float32) for _ in range(E // L)))
        for g in range(E // L):
          out_v[i, pl.ds(g * L, L)] = accs[g] / ln
        nxt = i + K

        @pl.when(nxt < NG)
        def _(b=b, nxt=nxt):
          pltpu.async_copy(table_hbm.at[idx_v.at[nxt]], rows[b], sems[b])
      return carry

    lax.fori_loop(0, NG // K, outer, 0)
    pltpu.sync_copy(out_v, out_hbm.at[pl.ds(wid * RPT, RPT)])

  return body


def kernel(k, lengths, table):
  B, H = k.shape
  V, E = table.shape
  return _build(B, H, V, E)(k, lengths, table)
